# baseline (speedup 1.0000x reference)
import jax
import jax.numpy as jnp
import numpy as np
from jax.experimental import pallas as pl

_H = 14
_P = 196
_B = 4
_CIN = 768
_CMID = 384
_COUT = 6
_FOLD = 24
_LW = _CIN // _FOLD
_DN = (((0,), (0,)), ((), ()))


def _conv_kernel(x_ref, wt_ref, b1_ref, w2_ref, b2_ref, o_ref):
    w = jax.lax.broadcasted_iota(jnp.int32, (_P, 1), 0) % _H
    for b in range(_B):
        xr = x_ref[b].astype(jnp.bfloat16)
        xb = jnp.concatenate(
            [xr[:, _P * j:_P * (j + 1)] for j in range(_LW)],
            axis=0)
        acc = jnp.zeros((_P, _CMID), dtype=jnp.float32)
        for dh in range(3):
            for dw in range(3):
                offr = (dh - 1) * _H + (dw - 1)
                full = jax.lax.dot_general(
                    xb, wt_ref[dh * 3 + dw], _DN,
                    preferred_element_type=jnp.float32)
                shifted = jax.lax.slice(
                    jnp.pad(full, ((15, 15), (0, 0))),
                    (15 + offr, 0), (15 + offr + _P, _CMID))
                if dw == 0:
                    shifted = jnp.where(w == 0, 0.0, shifted)
                elif dw == 2:
                    shifted = jnp.where(w == _H - 1, 0.0, shifted)
                acc = acc + shifted
        a = jnp.maximum(acc + b1_ref[...], 0.0).astype(jnp.bfloat16)
        out = jnp.dot(a, w2_ref[...], preferred_element_type=jnp.float32)
        o_ref[b] = (out + b2_ref[...]).T


def kernel(x, W1, b1, W2, b2):
    n = x.shape[0]
    xv = x.reshape(n, _FOLD, _LW * _P)
    cp = np.arange(_CIN)
    perm = _LW * (cp % _FOLD) + cp // _FOLD
    wt = jnp.transpose(W1, (2, 3, 1, 0)).reshape(9, _CIN, _CMID)
    wt = wt[:, perm, :].astype(jnp.bfloat16)
    w2 = W2.reshape(_COUT, _CMID).T.astype(jnp.bfloat16)
    b1r = b1.reshape(1, _CMID)
    b2r = b2.reshape(1, _COUT)

    out = pl.pallas_call(
        _conv_kernel,
        grid=(n // _B,),
        in_specs=[
            pl.BlockSpec((_B, _FOLD, _LW * _P), lambda i: (i, 0, 0)),
            pl.BlockSpec((9, _CIN, _CMID), lambda i: (0, 0, 0)),
            pl.BlockSpec((1, _CMID), lambda i: (0, 0)),
            pl.BlockSpec((_CMID, _COUT), lambda i: (0, 0)),
            pl.BlockSpec((1, _COUT), lambda i: (0, 0)),
        ],
        out_specs=pl.BlockSpec((_B, _COUT, _P), lambda i: (i, 0, 0)),
        out_shape=jax.ShapeDtypeStruct((n, _COUT, _P), jnp.float32),
    )(xv, wt, b1r, w2, b2r)

    return out.reshape(n, -1)


# speedup vs baseline: 2.5341x; 2.5341x over previous
import jax
import jax.numpy as jnp
from jax.experimental import pallas as pl
from jax.experimental.pallas import tpu as pltpu

_H = 14
_P = 196
_B = 8
_SEG = 232
_TOP = 16
_SROWS = _TOP + _B * _SEG + 16
_M = _B * _SEG
_CIN = 768
_CMID = 384
_COUT = 6


def _conv_kernel(x_ref, wt_ref, b1_ref, w2_ref, b2_ref, o_ref, s_ref):
    @pl.when(pl.program_id(0) == 0)
    def _init():
        s_ref[...] = jnp.zeros((_SROWS, _CIN), dtype=jnp.bfloat16)

    for b in range(_B):
        xt = x_ref[b].astype(jnp.bfloat16).T
        s_ref[_TOP + b * _SEG:_TOP + b * _SEG + _P, :] = xt

    w = (jax.lax.broadcasted_iota(jnp.int32, (_M, 1), 0) % _SEG) % _H
    acc = jnp.zeros((_M, _CMID), dtype=jnp.float32)
    for dh in range(3):
        for dw in range(3):
            offr = (dh - 1) * _H + (dw - 1)
            lhs = s_ref[_TOP + offr:_TOP + offr + _M, :]
            full = jnp.dot(lhs, wt_ref[dh * 3 + dw],
                           preferred_element_type=jnp.float32)
            if dw == 0:
                full = jnp.where(w == 0, 0.0, full)
            elif dw == 2:
                full = jnp.where(w == _H - 1, 0.0, full)
            acc = acc + full
    a = jnp.maximum(acc + b1_ref[...], 0.0).astype(jnp.bfloat16)
    out = jnp.dot(a, w2_ref[...], preferred_element_type=jnp.float32)
    out = out + b2_ref[...]
    for b in range(_B):
        o_ref[b] = out[b * _SEG:b * _SEG + _P, :].T


def kernel(x, W1, b1, W2, b2):
    n = x.shape[0]
    xv = x.reshape(n, _CIN, _P)
    wt = jnp.transpose(W1, (2, 3, 1, 0)).reshape(9, _CIN, _CMID)
    wt = wt.astype(jnp.bfloat16)
    w2 = W2.reshape(_COUT, _CMID).T.astype(jnp.bfloat16)
    b1r = b1.reshape(1, _CMID)
    b2r = b2.reshape(1, _COUT)

    out = pl.pallas_call(
        _conv_kernel,
        grid=(n // _B,),
        in_specs=[
            pl.BlockSpec((_B, _CIN, _P), lambda i: (i, 0, 0)),
            pl.BlockSpec((9, _CIN, _CMID), lambda i: (0, 0, 0)),
            pl.BlockSpec((1, _CMID), lambda i: (0, 0)),
            pl.BlockSpec((_CMID, _COUT), lambda i: (0, 0)),
            pl.BlockSpec((1, _COUT), lambda i: (0, 0)),
        ],
        out_specs=pl.BlockSpec((_B, _COUT, _P), lambda i: (i, 0, 0)),
        out_shape=jax.ShapeDtypeStruct((n, _COUT, _P), jnp.float32),
        scratch_shapes=[pltpu.VMEM((_SROWS, _CIN), jnp.bfloat16)],
    )(xv, wt, b1r, w2, b2r)

    return out.reshape(n, -1)
